# Initial kernel scaffold; baseline (speedup 1.0000x reference)
import jax
import jax.numpy as jnp
from jax.experimental import pallas as pl


def kernel(x, edge_index, W1, b1, W2, b2, W3, b3):
    raise NotImplementedError("write your pallas kernel here")


# speedup vs baseline: 6.1985x; 6.1985x over previous
import functools

import jax
import jax.numpy as jnp
from jax import lax
from jax.experimental import pallas as pl
from jax.experimental.pallas import tpu as pltpu
from jax.experimental.pallas import tpu_sc as plsc

NC = 2
NS = 16
KE = 128
DC = 128
NR = 10240
BM = 2000


def _mesh():
    return plsc.VectorSubcoreMesh(core_axis_name="c", subcore_axis_name="s")


def _sc_degree(dst_pad, n_nodes):
    epad = dst_pad.shape[0]
    per_sc = epad // NC
    per_tile = per_sc // NS
    n_chunks = per_tile // KE
    rpt = NR // NS

    def body(dst_hbm, out0, out1, acc, zb, ones, dstv):
        c = lax.axis_index("c")
        s = lax.axis_index("s")
        zero16 = jnp.zeros((16,), jnp.float32)
        one16 = jnp.ones((16,), jnp.float32)
        for i in range(16):
            zb[i, :] = zero16
        for i in range(KE):
            ones[i, :] = one16

        @pl.loop(0, rpt // 16)
        def _(i):
            pltpu.sync_copy(zb, acc.at[pl.ds(s * rpt + i * 16, 16)])

        plsc.subcore_barrier()

        base = c * per_sc + s * per_tile

        @pl.loop(0, n_chunks)
        def _(i):
            pltpu.sync_copy(dst_hbm.at[pl.ds(base + i * KE, KE)], dstv)
            pltpu.sync_copy(ones, acc.at[dstv], add=True)

        plsc.subcore_barrier()

        off = s * rpt

        @pl.when(c == 0)
        def _():
            pltpu.sync_copy(acc.at[pl.ds(off, rpt)], out0.at[pl.ds(off, rpt)])

        @pl.when(c == 1)
        def _():
            pltpu.sync_copy(acc.at[pl.ds(off, rpt)], out1.at[pl.ds(off, rpt)])

    out_t = jax.ShapeDtypeStruct((NR, 16), jnp.float32)
    f = pl.kernel(
        body,
        out_type=(out_t, out_t),
        mesh=_mesh(),
        scratch_types=(
            pltpu.VMEM_SHARED((NR, 16), jnp.float32),
            pltpu.VMEM((16, 16), jnp.float32),
            pltpu.VMEM((KE, 16), jnp.float32),
            pltpu.VMEM((KE,), jnp.int32),
        ),
    )
    return f(dst_pad)


def _sc_segsum(tables, src_pad, dst_pad, n_nodes):
    nch = len(tables)
    epad = src_pad.shape[0]
    per_tile = epad // NS
    n_chunks = per_tile // KE
    rpt = NR // NS

    def body(*refs):
        tabs = refs[:nch]
        src_hbm = refs[nch]
        dst_hbm = refs[nch + 1]
        outs = refs[nch + 2:2 * nch + 2]
        acc, zb, srcv, dstv, rows = refs[2 * nch + 2:]
        c = lax.axis_index("c")
        s = lax.axis_index("s")
        zero16 = jnp.zeros((16,), jnp.float32)
        for i in range(16):
            for j in range(DC // 16):
                zb[i, pl.ds(j * 16, 16)] = zero16

        base = s * per_tile
        for ch in range(nch):
            @pl.when(c == (ch % NC))
            def _(ch=ch):
                @pl.loop(0, rpt // 16)
                def _(i):
                    pltpu.sync_copy(zb, acc.at[pl.ds(s * rpt + i * 16, 16)])

                plsc.subcore_barrier()

                @pl.loop(0, n_chunks)
                def _(i):
                    off = base + i * KE
                    pltpu.sync_copy(src_hbm.at[pl.ds(off, KE)], srcv)
                    pltpu.sync_copy(dst_hbm.at[pl.ds(off, KE)], dstv)
                    pltpu.sync_copy(tabs[ch].at[srcv], rows)
                    pltpu.sync_copy(rows, acc.at[dstv], add=True)

                plsc.subcore_barrier()

                o = s * rpt
                pltpu.sync_copy(acc.at[pl.ds(o, rpt)],
                                outs[ch].at[pl.ds(o, rpt)])

                plsc.subcore_barrier()

    out_t = tuple(jax.ShapeDtypeStruct((NR, DC), jnp.float32)
                  for _ in range(nch))
    f = pl.kernel(
        body,
        out_type=out_t,
        mesh=_mesh(),
        scratch_types=(
            pltpu.VMEM_SHARED((NR, DC), jnp.float32),
            pltpu.VMEM((16, DC), jnp.float32),
            pltpu.VMEM((KE,), jnp.int32),
            pltpu.VMEM((KE,), jnp.int32),
            pltpu.VMEM((KE, DC), jnp.float32),
        ),
    )
    return f(*tables, src_pad, dst_pad)


def _row_spec(width):
    return pl.BlockSpec((BM, width), lambda i: (i, 0))


def _full_spec(r, cdim):
    return pl.BlockSpec((r, cdim), lambda i: (0, 0))


def _tc_pre(x, c0, c1):
    n, d_in = x.shape

    def body(x_ref, c0_ref, c1_ref, xt0_ref, xt1_ref, dv8_ref):
        deg = 1.0 + c0_ref[:, 0:1] + c1_ref[:, 0:1]
        dinv = lax.rsqrt(deg)
        xt = x_ref[...] * dinv
        xt0_ref[...] = xt[:, :DC]
        xt1_ref[...] = xt[:, DC:]
        dv8_ref[...] = jnp.broadcast_to(dinv, (BM, 8))

    return pl.pallas_call(
        body,
        grid=(n // BM,),
        in_specs=[_row_spec(d_in), _row_spec(16), _row_spec(16)],
        out_specs=[_row_spec(DC), _row_spec(DC), _row_spec(8)],
        out_shape=[
            jax.ShapeDtypeStruct((n, DC), jnp.float32),
            jax.ShapeDtypeStruct((n, DC), jnp.float32),
            jax.ShapeDtypeStruct((n, 8), jnp.float32),
        ],
    )(x, c0, c1)


def _tc_layer1(a0, a1, x0, x1, dv8, w, b):
    n = x0.shape[0]
    hid = w.shape[1]

    def body(a0r, a1r, x0r, x1r, dvr, wr, br, o0, o1, o2, o3):
        dinv = dvr[:, 0:1]
        m = (jnp.concatenate([a0r[...], a1r[...]], axis=1)
             + jnp.concatenate([x0r[...], x1r[...]], axis=1)) * dinv
        h = jnp.maximum(
            jnp.dot(m, wr[...], preferred_element_type=jnp.float32) + br[...],
            0.0)
        z = h * dinv
        o0[...] = z[:, 0 * DC:1 * DC]
        o1[...] = z[:, 1 * DC:2 * DC]
        o2[...] = z[:, 2 * DC:3 * DC]
        o3[...] = z[:, 3 * DC:4 * DC]

    return pl.pallas_call(
        body,
        grid=(n // BM,),
        in_specs=[_row_spec(DC)] * 4 + [_row_spec(8),
                                        _full_spec(w.shape[0], hid),
                                        _full_spec(1, hid)],
        out_specs=[_row_spec(DC)] * 4,
        out_shape=[jax.ShapeDtypeStruct((n, DC), jnp.float32)] * 4,
    )(a0, a1, x0, x1, dv8, w, b)


def _tc_layer2(a, z, dv8, w2, b2, w3):
    n = z[0].shape[0]
    hid = w2.shape[1]
    out_d = w3.shape[1]

    def body(a0r, a1r, a2r, a3r, z0r, z1r, z2r, z3r, dvr, w2r, b2r, w3r,
             g0, g1):
        dinv = dvr[:, 0:1]
        u = (jnp.concatenate([a0r[...], a1r[...], a2r[...], a3r[...]], axis=1)
             + jnp.concatenate([z0r[...], z1r[...], z2r[...], z3r[...]],
                               axis=1)) * dinv
        h = jnp.maximum(
            jnp.dot(u, w2r[...], preferred_element_type=jnp.float32)
            + b2r[...], 0.0)
        z3 = h * dinv
        g = jnp.dot(z3, w3r[...], preferred_element_type=jnp.float32)
        g0[...] = g[:, :DC]
        g1[...] = g[:, DC:]

    return pl.pallas_call(
        body,
        grid=(n // BM,),
        in_specs=[_row_spec(DC)] * 8 + [_row_spec(8),
                                        _full_spec(hid, hid),
                                        _full_spec(1, hid),
                                        _full_spec(hid, out_d)],
        out_specs=[_row_spec(DC)] * 2,
        out_shape=[jax.ShapeDtypeStruct((n, DC), jnp.float32)] * 2,
    )(*a, *z, dv8, w2, b2, w3)


def _tc_final(a0, a1, g0, g1, dv8, b3):
    n = g0.shape[0]
    out_d = 2 * DC

    def body(a0r, a1r, g0r, g1r, dvr, br, out_ref):
        dinv = dvr[:, 0:1]
        agg = (jnp.concatenate([a0r[...], a1r[...]], axis=1)
               + jnp.concatenate([g0r[...], g1r[...]], axis=1))
        out_ref[...] = agg * dinv + br[...]

    return pl.pallas_call(
        body,
        grid=(n // BM,),
        in_specs=[_row_spec(DC)] * 4 + [_row_spec(8), _full_spec(1, out_d)],
        out_specs=_row_spec(out_d),
        out_shape=jax.ShapeDtypeStruct((n, out_d), jnp.float32),
    )(a0, a1, g0, g1, dv8, b3)


def kernel(x, edge_index, W1, b1, W2, b2, W3, b3):
    n = x.shape[0]
    e = edge_index.shape[1]
    quantum = NC * NS * KE
    epad = ((e + quantum - 1) // quantum) * quantum
    pad = epad - e
    src_pad = jnp.concatenate(
        [edge_index[0], jnp.zeros((pad,), jnp.int32)])
    dst_pad = jnp.concatenate(
        [edge_index[1], jnp.full((pad,), n, jnp.int32)])

    b1r = b1.reshape(1, -1)
    b2r = b2.reshape(1, -1)
    b3r = b3.reshape(1, -1)

    c0, c1 = _sc_degree(dst_pad, n)
    xt0, xt1, dv8 = _tc_pre(x, c0, c1)
    a10, a11 = _sc_segsum([xt0, xt1], src_pad, dst_pad, n)
    z2 = _tc_layer1(a10, a11, xt0, xt1, dv8, W1, b1r)
    a2 = _sc_segsum(list(z2), src_pad, dst_pad, n)
    g30, g31 = _tc_layer2(list(a2), list(z2), dv8, W2, b2r, W3)
    a30, a31 = _sc_segsum([g30, g31], src_pad, dst_pad, n)
    return _tc_final(a30, a31, g30, g31, dv8, b3r)


# speedup vs baseline: 6.2982x; 1.0161x over previous
import functools

import jax
import jax.numpy as jnp
from jax import lax
from jax.experimental import pallas as pl
from jax.experimental.pallas import tpu as pltpu
from jax.experimental.pallas import tpu_sc as plsc

NC = 2
NS = 16
KE = 128
DC = 128
NR = 10112
BM = 2000


def _mesh():
    return plsc.VectorSubcoreMesh(core_axis_name="c", subcore_axis_name="s")


def _sc_degree(dst2, n_nodes):
    n_chunks = dst2.shape[1] // KE
    per_sc_ch = n_chunks // NC
    rpt = NR // NS

    def body(dst_hbm, out0, out1, acc, zb, ones, dstv):
        c = lax.axis_index("c")
        s = lax.axis_index("s")
        zero16 = jnp.zeros((16,), jnp.float32)
        one16 = jnp.ones((16,), jnp.float32)
        for i in range(8):
            for j in range(DC // 16):
                zb[i, pl.ds(j * 16, 16)] = zero16
        for i in range(KE):
            for j in range(DC // 16):
                ones[i, pl.ds(j * 16, 16)] = one16

        @pl.loop(0, rpt // 8)
        def _(i):
            pltpu.sync_copy(zb, acc.at[pl.ds(s * rpt + i * 8, 8)])

        plsc.subcore_barrier()

        @pl.loop(0, per_sc_ch)
        def _(g):
            pltpu.sync_copy(
                dst_hbm.at[s, pl.ds((c * per_sc_ch + g) * KE, KE)], dstv)
            pltpu.sync_copy(ones, acc.at[dstv], add=True)

        plsc.subcore_barrier()

        off = s * rpt

        @pl.when(c == 0)
        def _():
            pltpu.sync_copy(acc.at[pl.ds(off, rpt)], out0.at[pl.ds(off, rpt)])

        @pl.when(c == 1)
        def _():
            pltpu.sync_copy(acc.at[pl.ds(off, rpt)], out1.at[pl.ds(off, rpt)])

    out_t = jax.ShapeDtypeStruct((NR, DC), jnp.float32)
    f = pl.kernel(
        body,
        out_type=(out_t, out_t),
        mesh=_mesh(),
        scratch_types=(
            pltpu.VMEM_SHARED((NR, DC), jnp.float32),
            pltpu.VMEM((8, DC), jnp.float32),
            pltpu.VMEM((KE, DC), jnp.float32),
            pltpu.VMEM((KE,), jnp.int32),
        ),
    )
    return f(dst2)


NBUF = 2


def _sc_segsum(tables, src2, dst2, n_nodes):
    nch = len(tables)
    n_chunks = src2.shape[1] // KE
    rpt = NR // NS

    def body(*refs):
        tabs = refs[:nch]
        src_hbm = refs[nch]
        dst_hbm = refs[nch + 1]
        outs = refs[nch + 2:2 * nch + 2]
        acc, zb = refs[2 * nch + 2:2 * nch + 4]
        srcv = refs[2 * nch + 4:2 * nch + 4 + NBUF]
        dstv = refs[2 * nch + 4 + NBUF:2 * nch + 4 + 2 * NBUF]
        rows = refs[2 * nch + 4 + 2 * NBUF:2 * nch + 4 + 3 * NBUF]
        rest = refs[2 * nch + 4 + 3 * NBUF:]
        isems = rest[0:NBUF]
        gsems = rest[NBUF:2 * NBUF]
        c = lax.axis_index("c")
        s = lax.axis_index("s")
        zero16 = jnp.zeros((16,), jnp.float32)
        for i in range(8):
            for j in range(DC // 16):
                zb[i, pl.ds(j * 16, 16)] = zero16

        for ch in range(nch):
            @pl.when(c == (ch % NC))
            def _(ch=ch):
                @pl.loop(0, rpt // 8)
                def _(i):
                    pltpu.sync_copy(zb, acc.at[pl.ds(s * rpt + i * 8, 8)])

                plsc.subcore_barrier()

                @pl.loop(0, n_chunks // NBUF)
                def _(g):
                    idesc = []
                    for r in range(NBUF):
                        off = (g * NBUF + r) * KE
                        idesc.append((
                            pltpu.async_copy(
                                src_hbm.at[s, pl.ds(off, KE)], srcv[r],
                                isems[r]),
                            pltpu.async_copy(
                                dst_hbm.at[s, pl.ds(off, KE)], dstv[r],
                                isems[r])))
                    gd = []
                    for r in range(NBUF):
                        idesc[r][0].wait()
                        idesc[r][1].wait()
                        gd.append(pltpu.async_copy(
                            tabs[ch].at[srcv[r]], rows[r], gsems[r]))
                    for r in range(NBUF):
                        gd[r].wait()
                        pltpu.sync_copy(rows[r], acc.at[dstv[r]], add=True)

                plsc.subcore_barrier()

                o = s * rpt
                pltpu.sync_copy(acc.at[pl.ds(o, rpt)],
                                outs[ch].at[pl.ds(o, rpt)])

                plsc.subcore_barrier()

    out_t = tuple(jax.ShapeDtypeStruct((NR, DC), jnp.float32)
                  for _ in range(nch))
    f = pl.kernel(
        body,
        out_type=out_t,
        mesh=_mesh(),
        scratch_types=(
            pltpu.VMEM_SHARED((NR, DC), jnp.float32),
            pltpu.VMEM((8, DC), jnp.float32),
        ) + tuple(pltpu.VMEM((KE,), jnp.int32) for _ in range(2 * NBUF))
          + tuple(pltpu.VMEM((KE, DC), jnp.float32) for _ in range(NBUF))
          + tuple(pltpu.SemaphoreType.DMA for _ in range(2 * NBUF)),
    )
    return f(*tables, src2, dst2)


def _row_spec(width):
    return pl.BlockSpec((BM, width), lambda i: (i, 0))


def _full_spec(r, cdim):
    return pl.BlockSpec((r, cdim), lambda i: (0, 0))


def _tc_pre(x, c0, c1):
    n, d_in = x.shape

    def body(x_ref, c0_ref, c1_ref, xt0_ref, xt1_ref, dv8_ref):
        deg = 1.0 + c0_ref[:, 0:1] + c1_ref[:, 0:1]
        dinv = lax.rsqrt(deg)
        xt = x_ref[...] * dinv
        xt0_ref[...] = xt[:, :DC]
        xt1_ref[...] = xt[:, DC:]
        dv8_ref[...] = jnp.broadcast_to(dinv, (BM, 8))

    return pl.pallas_call(
        body,
        grid=(n // BM,),
        in_specs=[_row_spec(d_in), _row_spec(DC), _row_spec(DC)],
        out_specs=[_row_spec(DC), _row_spec(DC), _row_spec(8)],
        out_shape=[
            jax.ShapeDtypeStruct((n, DC), jnp.float32),
            jax.ShapeDtypeStruct((n, DC), jnp.float32),
            jax.ShapeDtypeStruct((n, 8), jnp.float32),
        ],
    )(x, c0, c1)


def _tc_layer1(a0, a1, x0, x1, dv8, w, b):
    n = x0.shape[0]
    hid = w.shape[1]

    def body(a0r, a1r, x0r, x1r, dvr, wr, br, o0, o1, o2, o3):
        dinv = dvr[:, 0:1]
        m = (jnp.concatenate([a0r[...], a1r[...]], axis=1)
             + jnp.concatenate([x0r[...], x1r[...]], axis=1)) * dinv
        h = jnp.maximum(
            jnp.dot(m, wr[...], preferred_element_type=jnp.float32) + br[...],
            0.0)
        z = h * dinv
        o0[...] = z[:, 0 * DC:1 * DC]
        o1[...] = z[:, 1 * DC:2 * DC]
        o2[...] = z[:, 2 * DC:3 * DC]
        o3[...] = z[:, 3 * DC:4 * DC]

    return pl.pallas_call(
        body,
        grid=(n // BM,),
        in_specs=[_row_spec(DC)] * 4 + [_row_spec(8),
                                        _full_spec(w.shape[0], hid),
                                        _full_spec(1, hid)],
        out_specs=[_row_spec(DC)] * 4,
        out_shape=[jax.ShapeDtypeStruct((n, DC), jnp.float32)] * 4,
    )(a0, a1, x0, x1, dv8, w, b)


def _tc_layer2(a, z, dv8, w2, b2, w3):
    n = z[0].shape[0]
    hid = w2.shape[1]
    out_d = w3.shape[1]

    def body(a0r, a1r, a2r, a3r, z0r, z1r, z2r, z3r, dvr, w2r, b2r, w3r,
             g0, g1):
        dinv = dvr[:, 0:1]
        u = (jnp.concatenate([a0r[...], a1r[...], a2r[...], a3r[...]], axis=1)
             + jnp.concatenate([z0r[...], z1r[...], z2r[...], z3r[...]],
                               axis=1)) * dinv
        h = jnp.maximum(
            jnp.dot(u, w2r[...], preferred_element_type=jnp.float32)
            + b2r[...], 0.0)
        z3 = h * dinv
        g = jnp.dot(z3, w3r[...], preferred_element_type=jnp.float32)
        g0[...] = g[:, :DC]
        g1[...] = g[:, DC:]

    return pl.pallas_call(
        body,
        grid=(n // BM,),
        in_specs=[_row_spec(DC)] * 8 + [_row_spec(8),
                                        _full_spec(hid, hid),
                                        _full_spec(1, hid),
                                        _full_spec(hid, out_d)],
        out_specs=[_row_spec(DC)] * 2,
        out_shape=[jax.ShapeDtypeStruct((n, DC), jnp.float32)] * 2,
    )(*a, *z, dv8, w2, b2, w3)


def _tc_final(a0, a1, g0, g1, dv8, b3):
    n = g0.shape[0]
    out_d = 2 * DC

    def body(a0r, a1r, g0r, g1r, dvr, br, out_ref):
        dinv = dvr[:, 0:1]
        agg = (jnp.concatenate([a0r[...], a1r[...]], axis=1)
               + jnp.concatenate([g0r[...], g1r[...]], axis=1))
        out_ref[...] = agg * dinv + br[...]

    return pl.pallas_call(
        body,
        grid=(n // BM,),
        in_specs=[_row_spec(DC)] * 4 + [_row_spec(8), _full_spec(1, out_d)],
        out_specs=_row_spec(out_d),
        out_shape=jax.ShapeDtypeStruct((n, out_d), jnp.float32),
    )(a0, a1, g0, g1, dv8, b3)


def kernel(x, edge_index, W1, b1, W2, b2, W3, b3):
    n = x.shape[0]
    e = edge_index.shape[1]
    quantum = NC * NS * KE * NBUF
    epad = ((e + quantum - 1) // quantum) * quantum
    pad = epad - e
    per_tile = epad // NS
    src3 = jnp.concatenate(
        [edge_index[0], jnp.zeros((pad,), jnp.int32)]).reshape(NS, per_tile)
    dst3 = jnp.concatenate(
        [edge_index[1], jnp.full((pad,), n, jnp.int32)]).reshape(NS, per_tile)

    b1r = b1.reshape(1, -1)
    b2r = b2.reshape(1, -1)
    b3r = b3.reshape(1, -1)

    c0, c1 = _sc_degree(dst3, n)
    xt0, xt1, dv8 = _tc_pre(x, c0, c1)
    a10, a11 = _sc_segsum([xt0, xt1], src3, dst3, n)
    z2 = _tc_layer1(a10, a11, xt0, xt1, dv8, W1, b1r)
    a2 = _sc_segsum(list(z2), src3, dst3, n)
    g30, g31 = _tc_layer2(list(a2), list(z2), dv8, W2, b2r, W3)
    a30, a31 = _sc_segsum([g30, g31], src3, dst3, n)
    return _tc_final(a30, a31, g30, g31, dv8, b3r)


# speedup vs baseline: 6.3204x; 1.0035x over previous
import functools

import jax
import jax.numpy as jnp
from jax import lax
from jax.experimental import pallas as pl
from jax.experimental.pallas import tpu as pltpu
from jax.experimental.pallas import tpu_sc as plsc

NC = 2
NS = 16
KE = 128
DC = 128
NR = 10112
BM = 2000


def _mesh():
    return plsc.VectorSubcoreMesh(core_axis_name="c", subcore_axis_name="s")


def _sc_degree(dst2, n_nodes):
    n_chunks = dst2.shape[1] // KE
    per_sc_ch = n_chunks // NC
    rpt = NR // NS

    def body(dst_hbm, out0, out1, acc, zb, ones, dstv):
        c = lax.axis_index("c")
        s = lax.axis_index("s")
        zero16 = jnp.zeros((16,), jnp.float32)
        one16 = jnp.ones((16,), jnp.float32)
        for i in range(8):
            for j in range(DC // 16):
                zb[i, pl.ds(j * 16, 16)] = zero16
        for i in range(KE):
            for j in range(DC // 16):
                ones[i, pl.ds(j * 16, 16)] = one16

        @pl.loop(0, rpt // 8)
        def _(i):
            pltpu.sync_copy(zb, acc.at[pl.ds(s * rpt + i * 8, 8)])

        plsc.subcore_barrier()

        @pl.loop(0, per_sc_ch)
        def _(g):
            pltpu.sync_copy(
                dst_hbm.at[s, pl.ds((c * per_sc_ch + g) * KE, KE)], dstv)
            pltpu.sync_copy(ones, acc.at[dstv], add=True)

        plsc.subcore_barrier()

        off = s * rpt

        @pl.when(c == 0)
        def _():
            pltpu.sync_copy(acc.at[pl.ds(off, rpt)], out0.at[pl.ds(off, rpt)])

        @pl.when(c == 1)
        def _():
            pltpu.sync_copy(acc.at[pl.ds(off, rpt)], out1.at[pl.ds(off, rpt)])

    out_t = jax.ShapeDtypeStruct((NR, DC), jnp.float32)
    f = pl.kernel(
        body,
        out_type=(out_t, out_t),
        mesh=_mesh(),
        scratch_types=(
            pltpu.VMEM_SHARED((NR, DC), jnp.float32),
            pltpu.VMEM((8, DC), jnp.float32),
            pltpu.VMEM((KE, DC), jnp.float32),
            pltpu.VMEM((KE,), jnp.int32),
        ),
    )
    return f(dst2)


NBUF = 2


def _sc_segsum(tables, src2, dst2, n_nodes):
    nch = len(tables)
    n_chunks = src2.shape[1] // KE
    rpt = NR // NS

    def body(*refs):
        tabs = refs[:nch]
        src_hbm = refs[nch]
        dst_hbm = refs[nch + 1]
        outs = refs[nch + 2:2 * nch + 2]
        acc, zb = refs[2 * nch + 2:2 * nch + 4]
        srcv = refs[2 * nch + 4:2 * nch + 4 + NBUF]
        dstv = refs[2 * nch + 4 + NBUF:2 * nch + 4 + 2 * NBUF]
        rows = refs[2 * nch + 4 + 2 * NBUF:2 * nch + 4 + 3 * NBUF]
        rest = refs[2 * nch + 4 + 3 * NBUF:]
        isems = rest[0:NBUF]
        gsems = rest[NBUF:2 * NBUF]
        ssems = rest[2 * NBUF:3 * NBUF]
        c = lax.axis_index("c")
        s = lax.axis_index("s")
        zero16 = jnp.zeros((16,), jnp.float32)
        for i in range(8):
            for j in range(DC // 16):
                zb[i, pl.ds(j * 16, 16)] = zero16

        for ch in range(nch):
            @pl.when(c == (ch % NC))
            def _(ch=ch):
                @pl.loop(0, rpt // 8)
                def _(i):
                    pltpu.sync_copy(zb, acc.at[pl.ds(s * rpt + i * 8, 8)])

                plsc.subcore_barrier()

                @pl.loop(0, n_chunks // NBUF)
                def _(g):
                    idesc = []
                    for r in range(NBUF):
                        off = (g * NBUF + r) * KE
                        idesc.append((
                            pltpu.async_copy(
                                src_hbm.at[s, pl.ds(off, KE)], srcv[r],
                                isems[r]),
                            pltpu.async_copy(
                                dst_hbm.at[s, pl.ds(off, KE)], dstv[r],
                                isems[r])))
                    gd = []
                    for r in range(NBUF):
                        idesc[r][0].wait()
                        idesc[r][1].wait()
                        gd.append(pltpu.async_copy(
                            tabs[ch].at[srcv[r]], rows[r], gsems[r]))
                    sd = []
                    for r in range(NBUF):
                        gd[r].wait()
                        sd.append(pltpu.async_copy(
                            rows[r], acc.at[dstv[r]], ssems[r], add=True))
                    for d in sd:
                        d.wait()

                plsc.subcore_barrier()

                o = s * rpt
                pltpu.sync_copy(acc.at[pl.ds(o, rpt)],
                                outs[ch].at[pl.ds(o, rpt)])

                plsc.subcore_barrier()

    out_t = tuple(jax.ShapeDtypeStruct((NR, DC), jnp.float32)
                  for _ in range(nch))
    f = pl.kernel(
        body,
        out_type=out_t,
        mesh=_mesh(),
        scratch_types=(
            pltpu.VMEM_SHARED((NR, DC), jnp.float32),
            pltpu.VMEM((8, DC), jnp.float32),
        ) + tuple(pltpu.VMEM((KE,), jnp.int32) for _ in range(2 * NBUF))
          + tuple(pltpu.VMEM((KE, DC), jnp.float32) for _ in range(NBUF))
          + tuple(pltpu.SemaphoreType.DMA for _ in range(3 * NBUF)),
    )
    return f(*tables, src2, dst2)


def _row_spec(width):
    return pl.BlockSpec((BM, width), lambda i: (i, 0))


def _full_spec(r, cdim):
    return pl.BlockSpec((r, cdim), lambda i: (0, 0))


def _tc_pre(x, c0, c1):
    n, d_in = x.shape

    def body(x_ref, c0_ref, c1_ref, xt0_ref, xt1_ref, dv8_ref):
        deg = 1.0 + c0_ref[:, 0:1] + c1_ref[:, 0:1]
        dinv = lax.rsqrt(deg)
        xt = x_ref[...] * dinv
        xt0_ref[...] = xt[:, :DC]
        xt1_ref[...] = xt[:, DC:]
        dv8_ref[...] = jnp.broadcast_to(dinv, (BM, 8))

    return pl.pallas_call(
        body,
        grid=(n // BM,),
        in_specs=[_row_spec(d_in), _row_spec(DC), _row_spec(DC)],
        out_specs=[_row_spec(DC), _row_spec(DC), _row_spec(8)],
        out_shape=[
            jax.ShapeDtypeStruct((n, DC), jnp.float32),
            jax.ShapeDtypeStruct((n, DC), jnp.float32),
            jax.ShapeDtypeStruct((n, 8), jnp.float32),
        ],
    )(x, c0, c1)


def _tc_layer1(a0, a1, x0, x1, dv8, w, b):
    n = x0.shape[0]
    hid = w.shape[1]

    def body(a0r, a1r, x0r, x1r, dvr, wr, br, o0, o1, o2, o3):
        dinv = dvr[:, 0:1]
        m = (jnp.concatenate([a0r[...], a1r[...]], axis=1)
             + jnp.concatenate([x0r[...], x1r[...]], axis=1)) * dinv
        h = jnp.maximum(
            jnp.dot(m, wr[...], preferred_element_type=jnp.float32) + br[...],
            0.0)
        z = h * dinv
        o0[...] = z[:, 0 * DC:1 * DC]
        o1[...] = z[:, 1 * DC:2 * DC]
        o2[...] = z[:, 2 * DC:3 * DC]
        o3[...] = z[:, 3 * DC:4 * DC]

    return pl.pallas_call(
        body,
        grid=(n // BM,),
        in_specs=[_row_spec(DC)] * 4 + [_row_spec(8),
                                        _full_spec(w.shape[0], hid),
                                        _full_spec(1, hid)],
        out_specs=[_row_spec(DC)] * 4,
        out_shape=[jax.ShapeDtypeStruct((n, DC), jnp.float32)] * 4,
    )(a0, a1, x0, x1, dv8, w, b)


def _tc_layer2(a, z, dv8, w2, b2, w3):
    n = z[0].shape[0]
    hid = w2.shape[1]
    out_d = w3.shape[1]

    def body(a0r, a1r, a2r, a3r, z0r, z1r, z2r, z3r, dvr, w2r, b2r, w3r,
             g0, g1):
        dinv = dvr[:, 0:1]
        u = (jnp.concatenate([a0r[...], a1r[...], a2r[...], a3r[...]], axis=1)
             + jnp.concatenate([z0r[...], z1r[...], z2r[...], z3r[...]],
                               axis=1)) * dinv
        h = jnp.maximum(
            jnp.dot(u, w2r[...], preferred_element_type=jnp.float32)
            + b2r[...], 0.0)
        z3 = h * dinv
        g = jnp.dot(z3, w3r[...], preferred_element_type=jnp.float32)
        g0[...] = g[:, :DC]
        g1[...] = g[:, DC:]

    return pl.pallas_call(
        body,
        grid=(n // BM,),
        in_specs=[_row_spec(DC)] * 8 + [_row_spec(8),
                                        _full_spec(hid, hid),
                                        _full_spec(1, hid),
                                        _full_spec(hid, out_d)],
        out_specs=[_row_spec(DC)] * 2,
        out_shape=[jax.ShapeDtypeStruct((n, DC), jnp.float32)] * 2,
    )(*a, *z, dv8, w2, b2, w3)


def _tc_final(a0, a1, g0, g1, dv8, b3):
    n = g0.shape[0]
    out_d = 2 * DC

    def body(a0r, a1r, g0r, g1r, dvr, br, out_ref):
        dinv = dvr[:, 0:1]
        agg = (jnp.concatenate([a0r[...], a1r[...]], axis=1)
               + jnp.concatenate([g0r[...], g1r[...]], axis=1))
        out_ref[...] = agg * dinv + br[...]

    return pl.pallas_call(
        body,
        grid=(n // BM,),
        in_specs=[_row_spec(DC)] * 4 + [_row_spec(8), _full_spec(1, out_d)],
        out_specs=_row_spec(out_d),
        out_shape=jax.ShapeDtypeStruct((n, out_d), jnp.float32),
    )(a0, a1, g0, g1, dv8, b3)


def kernel(x, edge_index, W1, b1, W2, b2, W3, b3):
    n = x.shape[0]
    e = edge_index.shape[1]
    quantum = NC * NS * KE * NBUF
    epad = ((e + quantum - 1) // quantum) * quantum
    pad = epad - e
    per_tile = epad // NS
    src3 = jnp.concatenate(
        [edge_index[0], jnp.zeros((pad,), jnp.int32)]).reshape(NS, per_tile)
    dst3 = jnp.concatenate(
        [edge_index[1], jnp.full((pad,), n, jnp.int32)]).reshape(NS, per_tile)

    b1r = b1.reshape(1, -1)
    b2r = b2.reshape(1, -1)
    b3r = b3.reshape(1, -1)

    c0, c1 = _sc_degree(dst3, n)
    xt0, xt1, dv8 = _tc_pre(x, c0, c1)
    a10, a11 = _sc_segsum([xt0, xt1], src3, dst3, n)
    z2 = _tc_layer1(a10, a11, xt0, xt1, dv8, W1, b1r)
    a2 = _sc_segsum(list(z2), src3, dst3, n)
    g30, g31 = _tc_layer2(list(a2), list(z2), dv8, W2, b2r, W3)
    a30, a31 = _sc_segsum([g30, g31], src3, dst3, n)
    return _tc_final(a30, a31, g30, g31, dv8, b3r)


# speedup vs baseline: 6.3290x; 1.0014x over previous
import jax
import jax.numpy as jnp
from jax import lax
from jax.experimental import pallas as pl
from jax.experimental.pallas import tpu as pltpu
from jax.experimental.pallas import tpu_sc as plsc

NC = 2
NS = 16
KE = 128
DC = 128
NR = 10112
BM = 2000


def _mesh():
    return plsc.VectorSubcoreMesh(core_axis_name="c", subcore_axis_name="s")


def _sc_degree(dst2, n_nodes):
    n_chunks = dst2.shape[1] // KE
    per_sc_ch = n_chunks // NC
    rpt = NR // NS

    def body(dst_hbm, out0, out1, acc, zb, ones, dstv):
        c = lax.axis_index("c")
        s = lax.axis_index("s")
        zero16 = jnp.zeros((16,), jnp.float32)
        one16 = jnp.ones((16,), jnp.float32)
        for i in range(8):
            for j in range(DC // 16):
                zb[i, pl.ds(j * 16, 16)] = zero16
        for i in range(KE):
            for j in range(DC // 16):
                ones[i, pl.ds(j * 16, 16)] = one16

        @pl.loop(0, rpt // 8)
        def _(i):
            pltpu.sync_copy(zb, acc.at[pl.ds(s * rpt + i * 8, 8)])

        plsc.subcore_barrier()

        @pl.loop(0, per_sc_ch)
        def _(g):
            pltpu.sync_copy(
                dst_hbm.at[s, pl.ds((c * per_sc_ch + g) * KE, KE)], dstv)
            pltpu.sync_copy(ones, acc.at[dstv], add=True)

        plsc.subcore_barrier()

        off = s * rpt

        @pl.when(c == 0)
        def _():
            pltpu.sync_copy(acc.at[pl.ds(off, rpt)], out0.at[pl.ds(off, rpt)])

        @pl.when(c == 1)
        def _():
            pltpu.sync_copy(acc.at[pl.ds(off, rpt)], out1.at[pl.ds(off, rpt)])

    out_t = jax.ShapeDtypeStruct((NR, DC), jnp.float32)
    f = pl.kernel(
        body,
        out_type=(out_t, out_t),
        mesh=_mesh(),
        scratch_types=(
            pltpu.VMEM_SHARED((NR, DC), jnp.float32),
            pltpu.VMEM((8, DC), jnp.float32),
            pltpu.VMEM((KE, DC), jnp.float32),
            pltpu.VMEM((KE,), jnp.int32),
        ),
    )
    return f(dst2)


NBUF = 2


def _sc_segsum(tables, src2, dst2, n_nodes):
    nch = len(tables)
    n_chunks = src2.shape[1] // KE
    rpt = NR // NS

    def body(*refs):
        tabs = refs[:nch]
        src_hbm = refs[nch]
        dst_hbm = refs[nch + 1]
        outs = refs[nch + 2:2 * nch + 2]
        acc, zb = refs[2 * nch + 2:2 * nch + 4]
        srcv = refs[2 * nch + 4:2 * nch + 4 + NBUF]
        dstv = refs[2 * nch + 4 + NBUF:2 * nch + 4 + 2 * NBUF]
        rows = refs[2 * nch + 4 + 2 * NBUF:2 * nch + 4 + 3 * NBUF]
        rest = refs[2 * nch + 4 + 3 * NBUF:]
        isems = rest[0:NBUF]
        gsems = rest[NBUF:2 * NBUF]
        ssems = rest[2 * NBUF:3 * NBUF]
        c = lax.axis_index("c")
        s = lax.axis_index("s")
        zero16 = jnp.zeros((16,), jnp.float32)
        for i in range(8):
            for j in range(DC // 16):
                zb[i, pl.ds(j * 16, 16)] = zero16

        for ch in range(nch):
            @pl.when(c == (ch % NC))
            def _(ch=ch):
                @pl.loop(0, rpt // 8)
                def _(i):
                    pltpu.sync_copy(zb, acc.at[pl.ds(s * rpt + i * 8, 8)])

                plsc.subcore_barrier()

                @pl.loop(0, n_chunks // NBUF)
                def _(g):
                    idesc = []
                    for r in range(NBUF):
                        off = (g * NBUF + r) * KE
                        idesc.append((
                            pltpu.async_copy(
                                src_hbm.at[s, pl.ds(off, KE)], srcv[r],
                                isems[r]),
                            pltpu.async_copy(
                                dst_hbm.at[s, pl.ds(off, KE)], dstv[r],
                                isems[r])))
                    gd = []
                    for r in range(NBUF):
                        idesc[r][0].wait()
                        idesc[r][1].wait()
                        gd.append(pltpu.async_copy(
                            tabs[ch].at[srcv[r]], rows[r], gsems[r]))
                    sd = []
                    for r in range(NBUF):
                        gd[r].wait()
                        sd.append(pltpu.async_copy(
                            rows[r], acc.at[dstv[r]], ssems[r], add=True))
                    for d in sd:
                        d.wait()

                plsc.subcore_barrier()

                o = s * rpt
                pltpu.sync_copy(acc.at[pl.ds(o, rpt)],
                                outs[ch].at[pl.ds(o, rpt)])

                plsc.subcore_barrier()

    out_t = tuple(jax.ShapeDtypeStruct((NR, DC), jnp.float32)
                  for _ in range(nch))
    f = pl.kernel(
        body,
        out_type=out_t,
        mesh=_mesh(),
        scratch_types=(
            pltpu.VMEM_SHARED((NR, DC), jnp.float32),
            pltpu.VMEM((8, DC), jnp.float32),
        ) + tuple(pltpu.VMEM((KE,), jnp.int32) for _ in range(2 * NBUF))
          + tuple(pltpu.VMEM((KE, DC), jnp.float32) for _ in range(NBUF))
          + tuple(pltpu.SemaphoreType.DMA for _ in range(3 * NBUF)),
    )
    return f(*tables, src2, dst2)


def _row_spec(width):
    return pl.BlockSpec((BM, width), lambda i: (i, 0))


def _full_spec(r, cdim):
    return pl.BlockSpec((r, cdim), lambda i: (0, 0))


def _tc_pre(x, c0, c1):
    n, d_in = x.shape

    def body(x_ref, c0_ref, c1_ref, xt0_ref, xt1_ref, dv8_ref):
        deg = 1.0 + c0_ref[:, 0:1] + c1_ref[:, 0:1]
        dinv = lax.rsqrt(deg)
        xt = x_ref[...] * dinv
        xt0_ref[...] = xt[:, :DC]
        xt1_ref[...] = xt[:, DC:]
        dv8_ref[...] = jnp.broadcast_to(dinv, (BM, 8))

    return pl.pallas_call(
        body,
        grid=(n // BM,),
        in_specs=[_row_spec(d_in), _row_spec(DC), _row_spec(DC)],
        out_specs=[_row_spec(DC), _row_spec(DC), _row_spec(8)],
        out_shape=[
            jax.ShapeDtypeStruct((n, DC), jnp.float32),
            jax.ShapeDtypeStruct((n, DC), jnp.float32),
            jax.ShapeDtypeStruct((n, 8), jnp.float32),
        ],
    )(x, c0, c1)


def _tc_layer1(a0, a1, x0, x1, dv8, w, b):
    n = x0.shape[0]
    hid = w.shape[1]

    def body(a0r, a1r, x0r, x1r, dvr, wr, br, o0, o1, o2, o3):
        dinv = dvr[:, 0:1]
        m = (jnp.concatenate([a0r[...], a1r[...]], axis=1)
             + jnp.concatenate([x0r[...], x1r[...]], axis=1)) * dinv
        h = jnp.maximum(
            jnp.dot(m, wr[...], preferred_element_type=jnp.float32) + br[...],
            0.0)
        z = h * dinv
        o0[...] = z[:, 0 * DC:1 * DC]
        o1[...] = z[:, 1 * DC:2 * DC]
        o2[...] = z[:, 2 * DC:3 * DC]
        o3[...] = z[:, 3 * DC:4 * DC]

    return pl.pallas_call(
        body,
        grid=(n // BM,),
        in_specs=[_row_spec(DC)] * 4 + [_row_spec(8),
                                        _full_spec(w.shape[0], hid),
                                        _full_spec(1, hid)],
        out_specs=[_row_spec(DC)] * 4,
        out_shape=[jax.ShapeDtypeStruct((n, DC), jnp.float32)] * 4,
    )(a0, a1, x0, x1, dv8, w, b)


def _tc_layer2(a, z, dv8, w2, b2, w3):
    n = z[0].shape[0]
    hid = w2.shape[1]
    out_d = w3.shape[1]

    def body(a0r, a1r, a2r, a3r, z0r, z1r, z2r, z3r, dvr, w2r, b2r, w3r,
             g0, g1):
        dinv = dvr[:, 0:1]
        u = (jnp.concatenate([a0r[...], a1r[...], a2r[...], a3r[...]], axis=1)
             + jnp.concatenate([z0r[...], z1r[...], z2r[...], z3r[...]],
                               axis=1)) * dinv
        h = jnp.maximum(
            jnp.dot(u, w2r[...], preferred_element_type=jnp.float32)
            + b2r[...], 0.0)
        z3 = h * dinv
        g = jnp.dot(z3, w3r[...], preferred_element_type=jnp.float32)
        g0[...] = g[:, :DC]
        g1[...] = g[:, DC:]

    return pl.pallas_call(
        body,
        grid=(n // BM,),
        in_specs=[_row_spec(DC)] * 8 + [_row_spec(8),
                                        _full_spec(hid, hid),
                                        _full_spec(1, hid),
                                        _full_spec(hid, out_d)],
        out_specs=[_row_spec(DC)] * 2,
        out_shape=[jax.ShapeDtypeStruct((n, DC), jnp.float32)] * 2,
    )(*a, *z, dv8, w2, b2, w3)


def _tc_final(a0, a1, g0, g1, dv8, b3):
    n = g0.shape[0]
    out_d = 2 * DC

    def body(a0r, a1r, g0r, g1r, dvr, br, out_ref):
        dinv = dvr[:, 0:1]
        agg = (jnp.concatenate([a0r[...], a1r[...]], axis=1)
               + jnp.concatenate([g0r[...], g1r[...]], axis=1))
        out_ref[...] = agg * dinv + br[...]

    return pl.pallas_call(
        body,
        grid=(n // BM,),
        in_specs=[_row_spec(DC)] * 4 + [_row_spec(8), _full_spec(1, out_d)],
        out_specs=_row_spec(out_d),
        out_shape=jax.ShapeDtypeStruct((n, out_d), jnp.float32),
    )(a0, a1, g0, g1, dv8, b3)


def kernel(x, edge_index, W1, b1, W2, b2, W3, b3):
    n = x.shape[0]
    e = edge_index.shape[1]
    quantum = NC * NS * KE * NBUF
    epad = ((e + quantum - 1) // quantum) * quantum
    pad = epad - e
    per_tile = epad // NS
    src3 = jnp.concatenate(
        [edge_index[0], jnp.zeros((pad,), jnp.int32)]).reshape(NS, per_tile)
    dst3 = jnp.concatenate(
        [edge_index[1], jnp.full((pad,), n, jnp.int32)]).reshape(NS, per_tile)

    b1r = b1.reshape(1, -1)
    b2r = b2.reshape(1, -1)
    b3r = b3.reshape(1, -1)

    c0, c1 = _sc_degree(dst3, n)
    xt0, xt1, dv8 = _tc_pre(x, c0, c1)
    a10, a11 = _sc_segsum([xt0, xt1], src3, dst3, n)
    z2 = _tc_layer1(a10, a11, xt0, xt1, dv8, W1, b1r)
    a2 = _sc_segsum(list(z2), src3, dst3, n)
    g30, g31 = _tc_layer2(list(a2), list(z2), dv8, W2, b2r, W3)
    a30, a31 = _sc_segsum([g30, g31], src3, dst3, n)
    return _tc_final(a30, a31, g30, g31, dv8, b3r)
